# Initial kernel scaffold; baseline (speedup 1.0000x reference)
import jax
import jax.numpy as jnp
from jax.experimental import pallas as pl


def kernel(x, edge_index):
    raise NotImplementedError("write your pallas kernel here")


# speedup vs baseline: 4.5064x; 4.5064x over previous
import functools

import jax
import jax.numpy as jnp
from jax import lax
from jax.experimental import pallas as pl
from jax.experimental.pallas import tpu as pltpu
from jax.experimental.pallas import tpu_sc as plsc

N_NODES = 10000
N_EDGES = 320000
D_FEAT = 128

NC = 2
NS = 16
NW = NC * NS

N_PAD = 10240
ROWS_PER_TILE = N_PAD // NS
CHUNK = 128
N_CHUNKS = N_EDGES // CHUNK
MAX_K = -(-N_CHUNKS // NW)

_mesh = plsc.VectorSubcoreMesh(core_axis_name="c", subcore_axis_name="s")


def _fill_1d(ref, n, value):
    def body(i, _):
        ref[pl.ds(i * 16, 16)] = jnp.full((16,), value, dtype=jnp.float32)
        return ()
    lax.fori_loop(0, n // 16, body, ())


def _deg_body(src_hbm, dst_hbm, hsrc_hbm, hdst_hbm,
              idx_s, idx_d, ones_v, zeros_v, hs_sh, hd_sh):
    cid = lax.axis_index("c")
    sid = lax.axis_index("s")
    wid = sid * NC + cid

    _fill_1d(ones_v, CHUNK, 1.0)
    _fill_1d(zeros_v, ROWS_PER_TILE, 0.0)
    pltpu.sync_copy(zeros_v, hs_sh.at[pl.ds(sid * ROWS_PER_TILE, ROWS_PER_TILE)])
    pltpu.sync_copy(zeros_v, hd_sh.at[pl.ds(sid * ROWS_PER_TILE, ROWS_PER_TILE)])
    plsc.subcore_barrier()

    def body(k, _):
        chunk = k * NW + wid

        @pl.when(chunk < N_CHUNKS)
        def _():
            base = chunk * CHUNK
            pltpu.sync_copy(src_hbm.at[pl.ds(base, CHUNK)], idx_s)
            pltpu.sync_copy(dst_hbm.at[pl.ds(base, CHUNK)], idx_d)
            pltpu.sync_copy(ones_v, hs_sh.at[idx_s], add=True)
            pltpu.sync_copy(ones_v, hd_sh.at[idx_d], add=True)
        return ()

    lax.fori_loop(0, MAX_K, body, ())
    plsc.subcore_barrier()

    sl = pl.ds(sid * ROWS_PER_TILE, ROWS_PER_TILE)
    pltpu.sync_copy(hs_sh.at[sl], hsrc_hbm.at[cid, sl])
    pltpu.sync_copy(hd_sh.at[sl], hdst_hbm.at[cid, sl])


_deg_call = pl.kernel(
    _deg_body,
    out_type=(
        jax.ShapeDtypeStruct((NC, N_PAD), jnp.float32),
        jax.ShapeDtypeStruct((NC, N_PAD), jnp.float32),
    ),
    mesh=_mesh,
    scratch_types=[
        pltpu.VMEM((CHUNK,), jnp.int32),
        pltpu.VMEM((CHUNK,), jnp.int32),
        pltpu.VMEM((CHUNK,), jnp.float32),
        pltpu.VMEM((ROWS_PER_TILE,), jnp.float32),
        pltpu.VMEM_SHARED((N_PAD,), jnp.float32),
        pltpu.VMEM_SHARED((N_PAD,), jnp.float32),
    ],
)


def _main_body(src_hbm, dst_hbm, y_hbm, part_hbm,
               idx_s, idx_d, rows_v, acc_sh, sem):
    cid = lax.axis_index("c")
    sid = lax.axis_index("s")
    wid = sid * NC + cid

    def zrow(i, _):
        def zlane(j, _):
            rows_v[i, pl.ds(j * 16, 16)] = jnp.zeros((16,), jnp.float32)
            return ()
        lax.fori_loop(0, D_FEAT // 16, zlane, ())
        return ()
    lax.fori_loop(0, CHUNK, zrow, ())

    def zacc(k, _):
        pltpu.sync_copy(
            rows_v, acc_sh.at[pl.ds(sid * ROWS_PER_TILE + k * CHUNK, CHUNK)])
        return ()
    lax.fori_loop(0, ROWS_PER_TILE // CHUNK, zacc, ())
    plsc.subcore_barrier()

    def body(k, _):
        chunk = k * NW + wid

        @pl.when(chunk < N_CHUNKS)
        def _():
            base = chunk * CHUNK
            pltpu.sync_copy(src_hbm.at[pl.ds(base, CHUNK)], idx_s)
            pltpu.sync_copy(dst_hbm.at[pl.ds(base, CHUNK)], idx_d)
            pltpu.async_copy(y_hbm.at[idx_s], rows_v, sem).wait()
            pltpu.sync_copy(rows_v, acc_sh.at[idx_d], add=True)
        return ()

    lax.fori_loop(0, MAX_K, body, ())
    plsc.subcore_barrier()

    def wb(k, _):
        sl = pl.ds(sid * ROWS_PER_TILE + k * CHUNK, CHUNK)
        pltpu.sync_copy(acc_sh.at[sl], part_hbm.at[cid, sl])
        return ()
    lax.fori_loop(0, ROWS_PER_TILE // CHUNK, wb, ())


_main_call = pl.kernel(
    _main_body,
    out_type=jax.ShapeDtypeStruct((NC, N_PAD, D_FEAT), jnp.float32),
    mesh=_mesh,
    scratch_types=[
        pltpu.VMEM((CHUNK,), jnp.int32),
        pltpu.VMEM((CHUNK,), jnp.int32),
        pltpu.VMEM((CHUNK, D_FEAT), jnp.float32),
        pltpu.VMEM_SHARED((N_PAD, D_FEAT), jnp.float32),
        pltpu.SemaphoreType.DMA,
    ],
)


def _prescale_body(x_ref, hs_ref, y_ref):
    deg = hs_ref[0] + hs_ref[1]
    norm = jax.lax.rsqrt(jnp.clip(deg, 1.0, None))
    y_ref[...] = x_ref[...] * norm


def _final_body(part_ref, hd_ref, out_ref):
    deg = hd_ref[0] + hd_ref[1]
    norm = jax.lax.rsqrt(jnp.clip(deg, 1.0, None))
    out_ref[...] = (part_ref[0] + part_ref[1]) * norm


_BLK = 1000


def _prescale_call(x, hs3):
    return pl.pallas_call(
        _prescale_body,
        grid=(N_NODES // _BLK,),
        in_specs=[
            pl.BlockSpec((_BLK, D_FEAT), lambda i: (i, 0)),
            pl.BlockSpec((NC, _BLK, 1), lambda i: (0, i, 0)),
        ],
        out_specs=pl.BlockSpec((_BLK, D_FEAT), lambda i: (i, 0)),
        out_shape=jax.ShapeDtypeStruct((N_NODES, D_FEAT), jnp.float32),
    )(x, hs3)


def _final_call(parts, hd3):
    return pl.pallas_call(
        _final_body,
        grid=(N_NODES // _BLK,),
        in_specs=[
            pl.BlockSpec((NC, _BLK, D_FEAT), lambda i: (0, i, 0)),
            pl.BlockSpec((NC, _BLK, 1), lambda i: (0, i, 0)),
        ],
        out_specs=pl.BlockSpec((_BLK, D_FEAT), lambda i: (i, 0)),
        out_shape=jax.ShapeDtypeStruct((N_NODES, D_FEAT), jnp.float32),
    )(parts, hd3)


def kernel(x, edge_index):
    src = edge_index[0].astype(jnp.int32)
    dst = edge_index[1].astype(jnp.int32)
    hsrc, hdst = _deg_call(src, dst)
    y = _prescale_call(x, hsrc.reshape(NC, N_PAD, 1))
    parts = _main_call(src, dst, y)
    return _final_call(parts, hdst.reshape(NC, N_PAD, 1))


# speedup vs baseline: 8.9006x; 1.9751x over previous
import functools

import jax
import jax.numpy as jnp
from jax import lax
from jax.experimental import pallas as pl
from jax.experimental.pallas import tpu as pltpu
from jax.experimental.pallas import tpu_sc as plsc

N_NODES = 10000
N_EDGES = 320000
D_FEAT = 128

NC = 2
NS = 16
NW = NC * NS

N_PAD = 10240
ROWS_PER_TILE = N_PAD // NS
ZCHUNK = 128

CHUNK = 80
K_PER_TILE = N_EDGES // NW // CHUNK

_mesh = plsc.VectorSubcoreMesh(core_axis_name="c", subcore_axis_name="s")


def _fill_1d(ref, n, value):
    def body(i, _):
        ref[pl.ds(i * 16, 16)] = jnp.full((16,), value, dtype=jnp.float32)
        return ()
    lax.fori_loop(0, n // 16, body, ())


def _deg_body(src_hbm, dst_hbm, hsrc_hbm, hdst_hbm,
              idx_s, idx_d, ones_v, zeros_v, hs_sh, hd_sh, sem):
    cid = lax.axis_index("c")
    sid = lax.axis_index("s")
    wid = sid * NC + cid

    pltpu.sync_copy(src_hbm.at[wid], idx_s)
    pltpu.sync_copy(dst_hbm.at[wid], idx_d)
    _fill_1d(ones_v, CHUNK, 1.0)
    _fill_1d(zeros_v, ROWS_PER_TILE, 0.0)
    pltpu.sync_copy(zeros_v, hs_sh.at[pl.ds(sid * ROWS_PER_TILE, ROWS_PER_TILE)])
    pltpu.sync_copy(zeros_v, hd_sh.at[pl.ds(sid * ROWS_PER_TILE, ROWS_PER_TILE)])
    plsc.subcore_barrier()

    W = 4

    def fire(k):
        pltpu.async_copy(ones_v, hs_sh.at[idx_s.at[k, 0]], sem, add=True)
        pltpu.async_copy(ones_v, hd_sh.at[idx_d.at[k, 0]], sem, add=True)

    def drain(k):
        pltpu.make_async_copy(ones_v, hs_sh.at[idx_s.at[k, 0]], sem).wait()
        pltpu.make_async_copy(ones_v, hd_sh.at[idx_d.at[k, 0]], sem).wait()

    def body(k, _):
        fire(k)

        @pl.when(k >= W)
        def _():
            drain(k - W)
        return ()

    lax.fori_loop(0, K_PER_TILE, body, ())

    def tail(k, _):
        drain(k)
        return ()
    lax.fori_loop(K_PER_TILE - W, K_PER_TILE, tail, ())
    plsc.subcore_barrier()

    sl = pl.ds(sid * ROWS_PER_TILE, ROWS_PER_TILE)
    pltpu.sync_copy(hs_sh.at[sl], hsrc_hbm.at[cid, sl])
    pltpu.sync_copy(hd_sh.at[sl], hdst_hbm.at[cid, sl])


_deg_call = pl.kernel(
    _deg_body,
    out_type=(
        jax.ShapeDtypeStruct((NC, N_PAD), jnp.float32),
        jax.ShapeDtypeStruct((NC, N_PAD), jnp.float32),
    ),
    mesh=_mesh,
    scratch_types=[
        pltpu.VMEM((K_PER_TILE, 1, CHUNK), jnp.int32),
        pltpu.VMEM((K_PER_TILE, 1, CHUNK), jnp.int32),
        pltpu.VMEM((CHUNK,), jnp.float32),
        pltpu.VMEM((ROWS_PER_TILE,), jnp.float32),
        pltpu.VMEM_SHARED((N_PAD,), jnp.float32),
        pltpu.VMEM_SHARED((N_PAD,), jnp.float32),
        pltpu.SemaphoreType.DMA,
    ],
)


def _main_body(src_hbm, dst_hbm, y_hbm, part_hbm,
               idx_s, idx_d, rows0, rows1, acc_sh, g0, g1):
    cid = lax.axis_index("c")
    sid = lax.axis_index("s")
    wid = sid * NC + cid

    ept = K_PER_TILE * CHUNK
    pltpu.sync_copy(src_hbm.at[pl.ds(wid * ept, ept)], idx_s)
    pltpu.sync_copy(dst_hbm.at[wid], idx_d)

    def sidx(k):
        return idx_s.at[pl.ds(k * CHUNK, CHUNK)]

    def zrow(i, _):
        def zlane(j, _):
            rows0[i, pl.ds(j * 16, 16)] = jnp.zeros((16,), jnp.float32)
            return ()
        lax.fori_loop(0, D_FEAT // 16, zlane, ())
        return ()
    lax.fori_loop(0, ZCHUNK, zrow, ())

    def zacc(k, _):
        pltpu.sync_copy(
            rows0.at[pl.ds(0, ZCHUNK)],
            acc_sh.at[pl.ds(sid * ROWS_PER_TILE + k * ZCHUNK, ZCHUNK)])
        return ()
    lax.fori_loop(0, ROWS_PER_TILE // ZCHUNK, zacc, ())
    plsc.subcore_barrier()

    pltpu.async_copy(y_hbm.at[sidx(0)], rows0, g0)

    def body(k2, _):
        ka = 2 * k2
        kb = ka + 1
        pltpu.async_copy(y_hbm.at[sidx(kb)], rows1, g1)
        pltpu.make_async_copy(y_hbm.at[sidx(ka)], rows0, g0).wait()
        pltpu.sync_copy(rows0, acc_sh.at[idx_d.at[ka, 0]], add=True)
        pltpu.async_copy(y_hbm.at[sidx(ka + 2)], rows0, g0)
        pltpu.make_async_copy(y_hbm.at[sidx(kb)], rows1, g1).wait()
        pltpu.sync_copy(rows1, acc_sh.at[idx_d.at[kb, 0]], add=True)
        return ()

    lax.fori_loop(0, K_PER_TILE // 2, body, ())
    klast = K_PER_TILE - 1
    pltpu.make_async_copy(y_hbm.at[sidx(klast)], rows0, g0).wait()
    pltpu.sync_copy(rows0, acc_sh.at[idx_d.at[klast, 0]], add=True)
    plsc.subcore_barrier()

    def wb(k, _):
        sl = pl.ds(sid * ROWS_PER_TILE + k * ZCHUNK, ZCHUNK)
        pltpu.sync_copy(acc_sh.at[sl], part_hbm.at[cid, sl])
        return ()
    lax.fori_loop(0, ROWS_PER_TILE // ZCHUNK, wb, ())


_main_call = pl.kernel(
    _main_body,
    out_type=jax.ShapeDtypeStruct((NC, N_PAD, D_FEAT), jnp.float32),
    mesh=_mesh,
    scratch_types=[
        pltpu.VMEM((K_PER_TILE * CHUNK,), jnp.int32),
        pltpu.VMEM((K_PER_TILE, 1, CHUNK), jnp.int32),
        pltpu.VMEM((CHUNK, D_FEAT), jnp.float32),
        pltpu.VMEM((CHUNK, D_FEAT), jnp.float32),
        pltpu.VMEM_SHARED((N_PAD, D_FEAT), jnp.float32),
        pltpu.SemaphoreType.DMA,
        pltpu.SemaphoreType.DMA,
    ],
)


def _prescale_body(x_ref, hs_ref, y_ref):
    deg = hs_ref[0] + hs_ref[1]
    norm = jax.lax.rsqrt(jnp.clip(deg, 1.0, None))
    y_ref[...] = x_ref[...] * norm


def _final_body(part_ref, hd_ref, out_ref):
    deg = hd_ref[0] + hd_ref[1]
    norm = jax.lax.rsqrt(jnp.clip(deg, 1.0, None))
    out_ref[...] = (part_ref[0] + part_ref[1]) * norm


_BLK = 1000


def _prescale_call(x, hs3):
    return pl.pallas_call(
        _prescale_body,
        grid=(N_NODES // _BLK,),
        in_specs=[
            pl.BlockSpec((_BLK, D_FEAT), lambda i: (i, 0)),
            pl.BlockSpec((NC, _BLK, 1), lambda i: (0, i, 0)),
        ],
        out_specs=pl.BlockSpec((_BLK, D_FEAT), lambda i: (i, 0)),
        out_shape=jax.ShapeDtypeStruct((N_NODES, D_FEAT), jnp.float32),
    )(x, hs3)


def _final_call(parts, hd3):
    return pl.pallas_call(
        _final_body,
        grid=(N_NODES // _BLK,),
        in_specs=[
            pl.BlockSpec((NC, _BLK, D_FEAT), lambda i: (0, i, 0)),
            pl.BlockSpec((NC, _BLK, 1), lambda i: (0, i, 0)),
        ],
        out_specs=pl.BlockSpec((_BLK, D_FEAT), lambda i: (i, 0)),
        out_shape=jax.ShapeDtypeStruct((N_NODES, D_FEAT), jnp.float32),
    )(parts, hd3)


def kernel(x, edge_index):
    src_flat = edge_index[0].astype(jnp.int32)
    src = src_flat.reshape(NW, K_PER_TILE, 1, CHUNK)
    dst = edge_index[1].astype(jnp.int32).reshape(NW, K_PER_TILE, 1, CHUNK)
    hsrc, hdst = _deg_call(src, dst)
    y = _prescale_call(x, hsrc.reshape(NC, N_PAD, 1))
    parts = _main_call(src_flat, dst, y)
    return _final_call(parts, hdst.reshape(NC, N_PAD, 1))


# speedup vs baseline: 9.9240x; 1.1150x over previous
import functools

import jax
import jax.numpy as jnp
from jax import lax
from jax.experimental import pallas as pl
from jax.experimental.pallas import tpu as pltpu
from jax.experimental.pallas import tpu_sc as plsc

N_NODES = 10000
N_EDGES = 320000
D_FEAT = 128

NC = 2
NS = 16
NW = NC * NS

N_PAD = 10240
ROWS_PER_TILE = N_PAD // NS
ZCHUNK = 128

CHUNK = 80
K_PER_TILE = N_EDGES // NW // CHUNK

_mesh = plsc.VectorSubcoreMesh(core_axis_name="c", subcore_axis_name="s")


def _fill_1d(ref, n, value):
    def body(i, _):
        ref[pl.ds(i * 16, 16)] = jnp.full((16,), value, dtype=jnp.float32)
        return ()
    lax.fori_loop(0, n // 16, body, ())


def _deg_body(src_hbm, dst_hbm, hsrc_hbm, hdst_hbm,
              idx_s, idx_d, ones_v, zeros_v, hs_sh, hd_sh, sem):
    cid = lax.axis_index("c")
    sid = lax.axis_index("s")
    wid = sid * NC + cid

    pltpu.sync_copy(src_hbm.at[wid], idx_s)
    pltpu.sync_copy(dst_hbm.at[wid], idx_d)
    _fill_1d(ones_v, CHUNK, 1.0)
    _fill_1d(zeros_v, ROWS_PER_TILE, 0.0)
    pltpu.sync_copy(zeros_v, hs_sh.at[pl.ds(sid * ROWS_PER_TILE, ROWS_PER_TILE)])
    pltpu.sync_copy(zeros_v, hd_sh.at[pl.ds(sid * ROWS_PER_TILE, ROWS_PER_TILE)])
    plsc.subcore_barrier()

    W = 4

    def fire(k):
        pltpu.async_copy(ones_v, hs_sh.at[idx_s.at[k, 0]], sem, add=True)
        pltpu.async_copy(ones_v, hd_sh.at[idx_d.at[k, 0]], sem, add=True)

    def drain(k):
        pltpu.make_async_copy(ones_v, hs_sh.at[idx_s.at[k, 0]], sem).wait()
        pltpu.make_async_copy(ones_v, hd_sh.at[idx_d.at[k, 0]], sem).wait()

    def body(k, _):
        fire(k)

        @pl.when(k >= W)
        def _():
            drain(k - W)
        return ()

    lax.fori_loop(0, K_PER_TILE, body, ())

    def tail(k, _):
        drain(k)
        return ()
    lax.fori_loop(K_PER_TILE - W, K_PER_TILE, tail, ())
    plsc.subcore_barrier()

    sl = pl.ds(sid * ROWS_PER_TILE, ROWS_PER_TILE)
    pltpu.sync_copy(hs_sh.at[sl], hsrc_hbm.at[cid, sl])
    pltpu.sync_copy(hd_sh.at[sl], hdst_hbm.at[cid, sl])


_deg_call = pl.kernel(
    _deg_body,
    out_type=(
        jax.ShapeDtypeStruct((NC, N_PAD), jnp.float32),
        jax.ShapeDtypeStruct((NC, N_PAD), jnp.float32),
    ),
    mesh=_mesh,
    scratch_types=[
        pltpu.VMEM((K_PER_TILE, 1, CHUNK), jnp.int32),
        pltpu.VMEM((K_PER_TILE, 1, CHUNK), jnp.int32),
        pltpu.VMEM((CHUNK,), jnp.float32),
        pltpu.VMEM((ROWS_PER_TILE,), jnp.float32),
        pltpu.VMEM_SHARED((N_PAD,), jnp.float32),
        pltpu.VMEM_SHARED((N_PAD,), jnp.float32),
        pltpu.SemaphoreType.DMA,
    ],
)


def _main_body(src_hbm, dst_hbm, y_hbm, part_hbm,
               idx_s, ring_d, rows0, rows1, rows2, acc_sh,
               g0, g1, g2, s0, s1, s2, dsem):
    cid = lax.axis_index("c")
    sid = lax.axis_index("s")
    wid = sid * NC + cid

    ept = K_PER_TILE * CHUNK
    pltpu.sync_copy(src_hbm.at[pl.ds(wid * ept, ept)], idx_s)

    def sidx(k):
        return idx_s.at[pl.ds(k * CHUNK, CHUNK)]

    def fire_ld(j):
        pltpu.async_copy(dst_hbm.at[wid, j], ring_d.at[j % 8], dsem)

    def wait_ld(j):
        pltpu.make_async_copy(dst_hbm.at[wid, j], ring_d.at[j % 8], dsem).wait()

    def zrow(i, _):
        def zlane(j, _):
            rows0[i, pl.ds(j * 16, 16)] = jnp.zeros((16,), jnp.float32)
            return ()
        lax.fori_loop(0, D_FEAT // 16, zlane, ())
        return ()
    lax.fori_loop(0, CHUNK, zrow, ())

    def zacc(k, _):
        pltpu.sync_copy(
            rows0.at[pl.ds(0, CHUNK)],
            acc_sh.at[pl.ds(sid * ROWS_PER_TILE + k * CHUNK, CHUNK)])
        return ()
    lax.fori_loop(0, ROWS_PER_TILE // CHUNK, zacc, ())
    plsc.subcore_barrier()

    def role(i, rows, gsem, ssem, prev_ssem, rows_nn, gsem_nn):
        @pl.when(i < K_PER_TILE)
        def _():
            pltpu.make_async_copy(y_hbm.at[sidx(i)], rows, gsem).wait()
            wait_ld(i)
            pltpu.async_copy(rows, acc_sh.at[ring_d.at[i % 8, 0]], ssem,
                             add=True)

            @pl.when(i >= 1)
            def _():
                pltpu.make_async_copy(
                    rows, acc_sh.at[ring_d.at[i % 8, 0]], prev_ssem).wait()

            @pl.when(i + 6 < K_PER_TILE)
            def _():
                fire_ld(i + 6)

            @pl.when(i + 2 < K_PER_TILE)
            def _():
                pltpu.async_copy(y_hbm.at[sidx(i + 2)], rows_nn, gsem_nn)

    for j in range(6):
        fire_ld(j)
    pltpu.async_copy(y_hbm.at[sidx(0)], rows0, g0)
    pltpu.async_copy(y_hbm.at[sidx(1)], rows1, g1)

    def body(k3, _):
        i = 3 * k3
        role(i, rows0, g0, s0, s2, rows2, g2)
        role(i + 1, rows1, g1, s1, s0, rows0, g0)
        role(i + 2, rows2, g2, s2, s1, rows1, g1)
        return ()

    lax.fori_loop(0, (K_PER_TILE + 2) // 3, body, ())
    pltpu.make_async_copy(
        rows1, acc_sh.at[ring_d.at[(K_PER_TILE - 1) % 8, 0]],
        s1, ).wait()
    plsc.subcore_barrier()

    def wb(k, _):
        sl = pl.ds(sid * ROWS_PER_TILE + k * ZCHUNK, ZCHUNK)
        pltpu.sync_copy(acc_sh.at[sl], part_hbm.at[cid, sl])
        return ()
    lax.fori_loop(0, ROWS_PER_TILE // ZCHUNK, wb, ())


_main_call = pl.kernel(
    _main_body,
    out_type=jax.ShapeDtypeStruct((NC, N_PAD, D_FEAT), jnp.float32),
    mesh=_mesh,
    scratch_types=[
        pltpu.VMEM((K_PER_TILE * CHUNK,), jnp.int32),
        pltpu.VMEM((8, 1, CHUNK), jnp.int32),
        pltpu.VMEM((CHUNK, D_FEAT), jnp.float32),
        pltpu.VMEM((CHUNK, D_FEAT), jnp.float32),
        pltpu.VMEM((CHUNK, D_FEAT), jnp.float32),
        pltpu.VMEM_SHARED((N_PAD, D_FEAT), jnp.float32),
        pltpu.SemaphoreType.DMA,
        pltpu.SemaphoreType.DMA,
        pltpu.SemaphoreType.DMA,
        pltpu.SemaphoreType.DMA,
        pltpu.SemaphoreType.DMA,
        pltpu.SemaphoreType.DMA,
        pltpu.SemaphoreType.DMA,
    ],
)


def _prescale_body(x_ref, hs_ref, y_ref):
    deg = hs_ref[0] + hs_ref[1]
    norm = jax.lax.rsqrt(jnp.clip(deg, 1.0, None))
    y_ref[...] = x_ref[...] * norm


def _final_body(part_ref, hd_ref, out_ref):
    deg = hd_ref[0] + hd_ref[1]
    norm = jax.lax.rsqrt(jnp.clip(deg, 1.0, None))
    out_ref[...] = (part_ref[0] + part_ref[1]) * norm


_BLK = 1000


def _prescale_call(x, hs3):
    return pl.pallas_call(
        _prescale_body,
        grid=(N_NODES // _BLK,),
        in_specs=[
            pl.BlockSpec((_BLK, D_FEAT), lambda i: (i, 0)),
            pl.BlockSpec((NC, _BLK, 1), lambda i: (0, i, 0)),
        ],
        out_specs=pl.BlockSpec((_BLK, D_FEAT), lambda i: (i, 0)),
        out_shape=jax.ShapeDtypeStruct((N_NODES, D_FEAT), jnp.float32),
    )(x, hs3)


def _final_call(parts, hd3):
    return pl.pallas_call(
        _final_body,
        grid=(N_NODES // _BLK,),
        in_specs=[
            pl.BlockSpec((NC, _BLK, D_FEAT), lambda i: (0, i, 0)),
            pl.BlockSpec((NC, _BLK, 1), lambda i: (0, i, 0)),
        ],
        out_specs=pl.BlockSpec((_BLK, D_FEAT), lambda i: (i, 0)),
        out_shape=jax.ShapeDtypeStruct((N_NODES, D_FEAT), jnp.float32),
    )(parts, hd3)


def kernel(x, edge_index):
    src_flat = edge_index[0].astype(jnp.int32)
    src = src_flat.reshape(NW, K_PER_TILE, 1, CHUNK)
    dst = edge_index[1].astype(jnp.int32).reshape(NW, K_PER_TILE, 1, CHUNK)
    hsrc, hdst = _deg_call(src, dst)
    y = _prescale_call(x, hsrc.reshape(NC, N_PAD, 1))
    parts = _main_call(src_flat, dst, y)
    return _final_call(parts, hdst.reshape(NC, N_PAD, 1))
